# Initial kernel scaffold; baseline (speedup 1.0000x reference)
import jax
import jax.numpy as jnp
from jax.experimental import pallas as pl


def kernel(center, offset, W1, b1, gamma, beta, W2, b2):
    raise NotImplementedError("write your pallas kernel here")


# speedup vs baseline: 1.5052x; 1.5052x over previous
import functools

import jax
import jax.numpy as jnp
from jax import lax
from jax.experimental import pallas as pl

KNN = 9
F = 11
OUT = 64
_H = lax.Precision.HIGHEST


def _dot(a, b, dims):
    return lax.dot_general(a, b, (dims, ((), ())),
                           preferred_element_type=jnp.float32, precision=_H)


def _knn_feat_body(n, bq, xyzT_ref, center_ref, rot01_ref, featT_ref, mom_ref, mu_ref):
    b = pl.program_id(0)
    xyzT = xyzT_ref[...]
    qT = xyzT_ref[:, pl.ds(b * bq, bq)]
    xc = [center_ref[:, c:c + 1] for c in range(3)]
    qr = [qT[c:c + 1, :] for c in range(3)]
    sqcol = (xc[0] * xc[0] + xc[1] * xc[1]) + xc[2] * xc[2]
    qsqrow = (qr[0] * qr[0] + qr[1] * qr[1]) + qr[2] * qr[2]
    dot = lax.dot_general(xyzT, qT, (((0,), (0,)), ((), ())),
                          preferred_element_type=jnp.float32)
    d = (qsqrow + sqcol) - 2.0 * dot

    gxs, gys, gzs = [], [], []
    for _ in range(KNN):
        ii = lax.broadcasted_iota(jnp.int32, (n, bq), 0)
        m = jnp.min(d, axis=0, keepdims=True)
        cand = jnp.where(d == m, ii, n)
        j = jnp.min(cand, axis=0, keepdims=True)
        onehot = ii == j
        d = jnp.where(onehot, jnp.inf, d)
        ohf = onehot.astype(jnp.float32)
        nbT = _dot(xyzT, ohf, ((1,), (0,)))
        gxs.append(nbT[0:1, :] - qT[0:1, :])
        gys.append(nbT[1:2, :] - qT[1:2, :])
        gzs.append(nbT[2:3, :] - qT[2:3, :])

    rot01 = rot01_ref[...]
    keys = []
    for t in range(KNN):
        G = jnp.concatenate([gxs[t], gys[t], gzs[t]], axis=0)
        xy = lax.dot_general(rot01, G, (((1,), (0,)), ((), ())),
                             preferred_element_type=jnp.float32)
        xp, yp = xy[0:1, :], xy[1:2, :]
        denom = jnp.abs(xp) + jnp.abs(yp)
        p = yp / denom
        key = jnp.where(xp < 0.0,
                        jnp.where(yp >= 0.0, 2.0 - p, -2.0 - p), p)
        keys.append(jnp.where(denom == 0.0, 0.0, key))

    ranks = []
    for t in range(KNN):
        acc = jnp.zeros_like(keys[0], dtype=jnp.int32)
        for u in range(KNN):
            if u == t:
                continue
            c = (keys[u] <= keys[t]) if u < t else (keys[u] < keys[t])
            acc = acc + c.astype(jnp.int32)
        ranks.append(acc)

    zero = jnp.zeros_like(gxs[0])
    sx = [zero] * KNN
    sy = [zero] * KNN
    sz = [zero] * KNN
    for t in range(KNN):
        for r in range(KNN):
            sel = ranks[t] == r
            sx[r] = sx[r] + jnp.where(sel, gxs[t], 0.0)
            sy[r] = sy[r] + jnp.where(sel, gys[t], 0.0)
            sz[r] = sz[r] + jnp.where(sel, gzs[t], 0.0)

    mom = jnp.zeros((F, F), jnp.float32)
    mus = jnp.zeros((F, 1), jnp.float32)
    for r in range(KNN):
        r2 = (r + 1) % KNN
        v1x, v1y, v1z = sx[r], sy[r], sz[r]
        v2x, v2y, v2z = sx[r2], sy[r2], sz[r2]
        crx = v1y * v2z - v1z * v2y
        cry = v1z * v2x - v1x * v2z
        crz = v1x * v2y - v1y * v2x
        nrm = jnp.sqrt(crx * crx + cry * cry + crz * crz)
        inv = 1.0 / jnp.maximum(nrm, 1e-12)
        nx, ny, nz = crx * inv, cry * inv, crz * inv
        cx = (v1x + v2x) / 3.0
        cy = (v1y + v2y) / 3.0
        cz = (v1z + v2z) / 3.0
        pos = nx * cx + ny * cy + nz * cz
        d01 = jnp.sqrt(v1x * v1x + v1y * v1y + v1z * v1z)
        ex, ey, ez = v2x - v1x, v2y - v1y, v2z - v1z
        d12 = jnp.sqrt(ex * ex + ey * ey + ez * ez)
        d20 = jnp.sqrt(v2x * v2x + v2y * v2y + v2z * v2z)
        area = 0.5 * nrm
        fr = jnp.concatenate(
            [nx, ny, nz, pos, cx, cy, cz, d01, d12, d20, area], axis=0)
        featT_ref[r] = fr
        mom = mom + _dot(fr, fr, ((1,), (1,)))
        mus = mus + jnp.sum(fr, axis=1, keepdims=True)

    @pl.when(b == 0)
    def _():
        mom_ref[...] = jnp.zeros_like(mom_ref)
        mu_ref[...] = jnp.zeros_like(mu_ref)

    mom_ref[...] += mom
    mu_ref[...] += mus


def _mlp_body(featT_ref, W1_ref, A_ref, C_ref, W2_ref, b2k_ref, out_ref):
    W1m = W1_ref[...]
    A = A_ref[...]
    C = C_ref[...]
    bn = out_ref.shape[0]
    s = jnp.zeros((OUT, bn), jnp.float32)
    for k in range(KNN):
        Fk = featT_ref[k]
        y = _dot(W1m, Fk, ((1,), (0,)))
        s = s + jnp.maximum(A * y + C, 0.0)
    out = _dot(s, W2_ref[...], ((0,), (1,)))
    out_ref[...] = out + b2k_ref[...]


@functools.partial(jax.jit, static_argnames=())
def kernel(center, offset, W1, b1, gamma, beta, W2, b2):
    del offset
    n = center.shape[0]
    bq = min(128, n)
    xyzT = jnp.zeros((8, n), jnp.float32).at[0:3, :].set(center.T)

    featT, mom, musum = pl.pallas_call(
        functools.partial(_knn_feat_body, n, bq),
        grid=(n // bq,),
        in_specs=[pl.BlockSpec((8, n), lambda b: (0, 0)),
                  pl.BlockSpec((n, 3), lambda b: (0, 0)),
                  pl.BlockSpec((2, 3), lambda b: (0, 0))],
        out_specs=[
            pl.BlockSpec((KNN, F, bq), lambda b: (0, 0, b)),
            pl.BlockSpec((F, F), lambda b: (0, 0)),
            pl.BlockSpec((F, 1), lambda b: (0, 0)),
        ],
        out_shape=[
            jax.ShapeDtypeStruct((KNN, F, n), jnp.float32),
            jax.ShapeDtypeStruct((F, F), jnp.float32),
            jax.ShapeDtypeStruct((F, 1), jnp.float32),
        ],
    )(xyzT, center,
      jnp.array([[0.5, 0.7071, -0.5], [-0.5, 0.7071, 0.5]], jnp.float32))

    S = jnp.float32(n * KNN)
    mu = musum[:, 0] / S
    Mc = mom / S - jnp.outer(mu, mu)
    varx = jnp.einsum('oc,cd,od->o', W1, Mc, W1, precision=_H)
    meanx = jnp.einsum('oc,c->o', W1, mu, precision=_H) + b1
    Avec = gamma / jnp.sqrt(varx + 1e-5)
    Cvec = Avec * (b1 - meanx) + beta

    bn = min(2048, n)
    out = pl.pallas_call(
        _mlp_body,
        grid=(n // bn,),
        in_specs=[
            pl.BlockSpec((KNN, F, bn), lambda b: (0, 0, b)),
            pl.BlockSpec((OUT, F), lambda b: (0, 0)),
            pl.BlockSpec((OUT, 1), lambda b: (0, 0)),
            pl.BlockSpec((OUT, 1), lambda b: (0, 0)),
            pl.BlockSpec((OUT, OUT), lambda b: (0, 0)),
            pl.BlockSpec((1, OUT), lambda b: (0, 0)),
        ],
        out_specs=pl.BlockSpec((bn, OUT), lambda b: (b, 0)),
        out_shape=jax.ShapeDtypeStruct((n, OUT), jnp.float32),
    )(featT, W1, Avec[:, None], Cvec[:, None], W2,
      (jnp.float32(KNN) * b2)[None, :])
    return out


# speedup vs baseline: 1.7793x; 1.1821x over previous
import functools

import jax
import jax.numpy as jnp
from jax import lax
from jax.experimental import pallas as pl

KNN = 9
F = 11
OUT = 64
_H = lax.Precision.HIGHEST


def _dot(a, b, dims):
    return lax.dot_general(a, b, (dims, ((), ())),
                           preferred_element_type=jnp.float32, precision=_H)


def _knn_feat_body(n, bq, xyzT_ref, center_ref, rot01_ref, featT_ref, mom_ref, mu_ref):
    b = pl.program_id(0)
    xyzT = xyzT_ref[...]
    qT = xyzT_ref[:, pl.ds(b * bq, bq)]
    xc = [center_ref[:, c:c + 1] for c in range(3)]
    qr = [qT[c:c + 1, :] for c in range(3)]
    sqcol = (xc[0] * xc[0] + xc[1] * xc[1]) + xc[2] * xc[2]
    qsqrow = (qr[0] * qr[0] + qr[1] * qr[1]) + qr[2] * qr[2]
    dot = lax.dot_general(xyzT, qT, (((0,), (0,)), ((), ())),
                          preferred_element_type=jnp.float32)
    d = (qsqrow + sqcol) - 2.0 * dot

    gxs, gys, gzs = [], [], []
    for _ in range(KNN):
        ii = lax.broadcasted_iota(jnp.int32, (n, bq), 0)
        j = jnp.argmin(d, axis=0, keepdims=True)
        onehot = ii == j
        d = jnp.where(onehot, jnp.inf, d)
        ohf = onehot.astype(jnp.float32)
        nbT = _dot(xyzT, ohf, ((1,), (0,)))
        gxs.append(nbT[0:1, :] - qT[0:1, :])
        gys.append(nbT[1:2, :] - qT[1:2, :])
        gzs.append(nbT[2:3, :] - qT[2:3, :])

    rot01 = rot01_ref[...]
    keys = []
    for t in range(KNN):
        G = jnp.concatenate([gxs[t], gys[t], gzs[t]], axis=0)
        xy = lax.dot_general(rot01, G, (((1,), (0,)), ((), ())),
                             preferred_element_type=jnp.float32)
        xp, yp = xy[0:1, :], xy[1:2, :]
        denom = jnp.abs(xp) + jnp.abs(yp)
        p = yp / denom
        key = jnp.where(xp < 0.0,
                        jnp.where(yp >= 0.0, 2.0 - p, -2.0 - p), p)
        keys.append(jnp.where(denom == 0.0, 0.0, key))

    ranks = []
    for t in range(KNN):
        acc = jnp.zeros_like(keys[0], dtype=jnp.int32)
        for u in range(KNN):
            if u == t:
                continue
            c = (keys[u] <= keys[t]) if u < t else (keys[u] < keys[t])
            acc = acc + c.astype(jnp.int32)
        ranks.append(acc)

    zero = jnp.zeros_like(gxs[0])
    sx = [zero] * KNN
    sy = [zero] * KNN
    sz = [zero] * KNN
    for t in range(KNN):
        for r in range(KNN):
            sel = ranks[t] == r
            sx[r] = sx[r] + jnp.where(sel, gxs[t], 0.0)
            sy[r] = sy[r] + jnp.where(sel, gys[t], 0.0)
            sz[r] = sz[r] + jnp.where(sel, gzs[t], 0.0)

    mom = jnp.zeros((F, F), jnp.float32)
    mus = jnp.zeros((F, 1), jnp.float32)
    for r in range(KNN):
        r2 = (r + 1) % KNN
        v1x, v1y, v1z = sx[r], sy[r], sz[r]
        v2x, v2y, v2z = sx[r2], sy[r2], sz[r2]
        crx = v1y * v2z - v1z * v2y
        cry = v1z * v2x - v1x * v2z
        crz = v1x * v2y - v1y * v2x
        nrm = jnp.sqrt(crx * crx + cry * cry + crz * crz)
        inv = 1.0 / jnp.maximum(nrm, 1e-12)
        nx, ny, nz = crx * inv, cry * inv, crz * inv
        cx = (v1x + v2x) / 3.0
        cy = (v1y + v2y) / 3.0
        cz = (v1z + v2z) / 3.0
        pos = nx * cx + ny * cy + nz * cz
        d01 = jnp.sqrt(v1x * v1x + v1y * v1y + v1z * v1z)
        ex, ey, ez = v2x - v1x, v2y - v1y, v2z - v1z
        d12 = jnp.sqrt(ex * ex + ey * ey + ez * ez)
        d20 = jnp.sqrt(v2x * v2x + v2y * v2y + v2z * v2z)
        area = 0.5 * nrm
        fr = jnp.concatenate(
            [nx, ny, nz, pos, cx, cy, cz, d01, d12, d20, area], axis=0)
        featT_ref[r] = fr
        mom = mom + _dot(fr, fr, ((1,), (1,)))
        mus = mus + jnp.sum(fr, axis=1, keepdims=True)

    @pl.when(b == 0)
    def _():
        mom_ref[...] = jnp.zeros_like(mom_ref)
        mu_ref[...] = jnp.zeros_like(mu_ref)

    mom_ref[...] += mom
    mu_ref[...] += mus


def _mlp_body(featT_ref, W1_ref, A_ref, C_ref, W2_ref, b2k_ref, out_ref):
    W1m = W1_ref[...]
    A = A_ref[...]
    C = C_ref[...]
    bn = out_ref.shape[0]
    s = jnp.zeros((OUT, bn), jnp.float32)
    for k in range(KNN):
        Fk = featT_ref[k]
        y = _dot(W1m, Fk, ((1,), (0,)))
        s = s + jnp.maximum(A * y + C, 0.0)
    out = _dot(s, W2_ref[...], ((0,), (1,)))
    out_ref[...] = out + b2k_ref[...]


@functools.partial(jax.jit, static_argnames=())
def kernel(center, offset, W1, b1, gamma, beta, W2, b2):
    del offset
    n = center.shape[0]
    bq = min(128, n)
    xyzT = jnp.zeros((8, n), jnp.float32).at[0:3, :].set(center.T)

    featT, mom, musum = pl.pallas_call(
        functools.partial(_knn_feat_body, n, bq),
        grid=(n // bq,),
        in_specs=[pl.BlockSpec((8, n), lambda b: (0, 0)),
                  pl.BlockSpec((n, 3), lambda b: (0, 0)),
                  pl.BlockSpec((2, 3), lambda b: (0, 0))],
        out_specs=[
            pl.BlockSpec((KNN, F, bq), lambda b: (0, 0, b)),
            pl.BlockSpec((F, F), lambda b: (0, 0)),
            pl.BlockSpec((F, 1), lambda b: (0, 0)),
        ],
        out_shape=[
            jax.ShapeDtypeStruct((KNN, F, n), jnp.float32),
            jax.ShapeDtypeStruct((F, F), jnp.float32),
            jax.ShapeDtypeStruct((F, 1), jnp.float32),
        ],
    )(xyzT, center,
      jnp.array([[0.5, 0.7071, -0.5], [-0.5, 0.7071, 0.5]], jnp.float32))

    S = jnp.float32(n * KNN)
    mu = musum[:, 0] / S
    Mc = mom / S - jnp.outer(mu, mu)
    varx = jnp.einsum('oc,cd,od->o', W1, Mc, W1, precision=_H)
    meanx = jnp.einsum('oc,c->o', W1, mu, precision=_H) + b1
    Avec = gamma / jnp.sqrt(varx + 1e-5)
    Cvec = Avec * (b1 - meanx) + beta

    bn = min(2048, n)
    out = pl.pallas_call(
        _mlp_body,
        grid=(n // bn,),
        in_specs=[
            pl.BlockSpec((KNN, F, bn), lambda b: (0, 0, b)),
            pl.BlockSpec((OUT, F), lambda b: (0, 0)),
            pl.BlockSpec((OUT, 1), lambda b: (0, 0)),
            pl.BlockSpec((OUT, 1), lambda b: (0, 0)),
            pl.BlockSpec((OUT, OUT), lambda b: (0, 0)),
            pl.BlockSpec((1, OUT), lambda b: (0, 0)),
        ],
        out_specs=pl.BlockSpec((bn, OUT), lambda b: (b, 0)),
        out_shape=jax.ShapeDtypeStruct((n, OUT), jnp.float32),
    )(featT, W1, Avec[:, None], Cvec[:, None], W2,
      (jnp.float32(KNN) * b2)[None, :])
    return out
